# Initial kernel scaffold; baseline (speedup 1.0000x reference)
import jax
import jax.numpy as jnp
from jax.experimental import pallas as pl


def kernel(z, mask):
    raise NotImplementedError("write your pallas kernel here")


# speedup vs baseline: 6.7660x; 6.7660x over previous
import functools

import jax
import jax.numpy as jnp
from jax import lax
from jax.experimental import pallas as pl
from jax.experimental.pallas import tpu as pltpu

BATCH = 16384
Z_DIM = 4096
N_BIT = 1024
ROW_BLOCK = 256
MASK_ROWS = 8
MASK_COLS = N_BIT // MASK_ROWS


def _mask_kernel(mask_ref, z_ref, out_ref, keep_ref):
    @pl.when(pl.program_id(0) == 0)
    def _build_keep():
        cols = lax.broadcasted_iota(jnp.int32, (MASK_COLS, Z_DIM), 1)

        def body(k, keep):
            m = mask_ref[k, :].reshape(MASK_COLS, 1)
            hit = jnp.any(m == cols, axis=0, keepdims=True)
            return keep * (1.0 - hit.astype(jnp.float32))

        keep_ref[...] = lax.fori_loop(
            0, MASK_ROWS, body, jnp.ones((1, Z_DIM), jnp.float32))

    out_ref[...] = z_ref[...] * keep_ref[...]


@jax.jit
def kernel(z, mask):
    mask2d = mask.reshape(MASK_ROWS, MASK_COLS)
    grid = (BATCH // ROW_BLOCK,)
    return pl.pallas_call(
        _mask_kernel,
        grid=grid,
        in_specs=[
            pl.BlockSpec((MASK_ROWS, MASK_COLS), lambda i: (0, 0)),
            pl.BlockSpec((ROW_BLOCK, Z_DIM), lambda i: (i, 0)),
        ],
        out_specs=pl.BlockSpec((ROW_BLOCK, Z_DIM), lambda i: (i, 0)),
        out_shape=jax.ShapeDtypeStruct((BATCH, Z_DIM), jnp.float32),
        scratch_shapes=[pltpu.VMEM((1, Z_DIM), jnp.float32)],
        compiler_params=pltpu.CompilerParams(
            dimension_semantics=("arbitrary",),
        ),
    )(mask2d, z)


# speedup vs baseline: 6.8746x; 1.0161x over previous
import functools

import jax
import jax.numpy as jnp
from jax import lax
from jax.experimental import pallas as pl
from jax.experimental.pallas import tpu as pltpu

BATCH = 16384
Z_DIM = 4096
N_BIT = 1024
ROW_BLOCK = 512
MASK_ROWS = 8
MASK_COLS = N_BIT // MASK_ROWS


def _mask_kernel(mask_ref, z_ref, out_ref, keep_ref):
    @pl.when(pl.program_id(0) == 0)
    def _build_keep():
        cols = lax.broadcasted_iota(jnp.int32, (MASK_COLS, Z_DIM), 1)

        def body(k, keep):
            m = mask_ref[k, :].reshape(MASK_COLS, 1)
            hit = jnp.any(m == cols, axis=0, keepdims=True)
            return keep * (1.0 - hit.astype(jnp.float32))

        keep_ref[...] = lax.fori_loop(
            0, MASK_ROWS, body, jnp.ones((1, Z_DIM), jnp.float32))

    out_ref[...] = z_ref[...] * keep_ref[...]


@jax.jit
def kernel(z, mask):
    mask2d = mask.reshape(MASK_ROWS, MASK_COLS)
    grid = (BATCH // ROW_BLOCK,)
    return pl.pallas_call(
        _mask_kernel,
        grid=grid,
        in_specs=[
            pl.BlockSpec((MASK_ROWS, MASK_COLS), lambda i: (0, 0)),
            pl.BlockSpec((ROW_BLOCK, Z_DIM), lambda i: (i, 0)),
        ],
        out_specs=pl.BlockSpec((ROW_BLOCK, Z_DIM), lambda i: (i, 0)),
        out_shape=jax.ShapeDtypeStruct((BATCH, Z_DIM), jnp.float32),
        scratch_shapes=[pltpu.VMEM((1, Z_DIM), jnp.float32)],
        compiler_params=pltpu.CompilerParams(
            dimension_semantics=("arbitrary",),
        ),
    )(mask2d, z)
